# baseline (speedup 1.0000x reference)
import functools

import jax
import jax.numpy as jnp
from jax import lax
from jax.experimental import pallas as pl
from jax.experimental.pallas import tpu as pltpu
from jax.experimental.pallas import tpu_sc as plsc

_B = 4096
_L = 50
_DIM = 64
_N = _B * _L
_V = 1000000

_TBLK = 16384
_NBLK = (_V + _TBLK - 1) // _TBLK
_VP = _NBLK * (_TBLK // 2)

_NW = 32
_BW = _B // _NW


def _tc_prep_body(x_ref, w_ref, b_ref, o_ref):
    x = x_ref[...]
    cdims = (((0,), (1,)), ((), ()))
    a = lax.dot_general(x[:, :_TBLK // 2], w_ref[...], cdims,
                        preferred_element_type=jnp.float32)
    c = lax.dot_general(x[:, _TBLK // 2:], w_ref[...], cdims,
                        preferred_element_type=jnp.float32)
    bb = b_ref[...]
    o_ref[...] = jnp.concatenate([a + bb, c + bb], axis=1)


_tc_prep = pl.pallas_call(
    _tc_prep_body,
    grid=(_NBLK,),
    in_specs=[
        pl.BlockSpec((_DIM, _TBLK), lambda i: (0, i)),
        pl.BlockSpec((_DIM, _DIM), lambda i: (0, 0)),
        pl.BlockSpec((1, _DIM), lambda i: (0, 0)),
    ],
    out_specs=pl.BlockSpec((_TBLK // 2, 2 * _DIM), lambda i: (i, 0)),
    out_shape=jax.ShapeDtypeStruct((_VP, 2 * _DIM), jnp.float32),
)


def _sc_body(idx_hbm, par_hbm, msk_hbm, table_hbm, out_hbm,
             idx_v, par_v, msk_v, buf_a, buf_b, outb, sem_a, sem_b):
    wid = lax.axis_index("s") * 2 + lax.axis_index("c")
    col0 = wid * _BW
    pltpu.sync_copy(idx_hbm.at[:, pl.ds(col0, _BW)], idx_v)
    pltpu.sync_copy(par_hbm.at[:, pl.ds(col0, _BW)], par_v)
    pltpu.sync_copy(msk_hbm.at[:, pl.ds(col0, _BW)], msk_v)

    bufs = (buf_a, buf_b)
    sems = (sem_a, sem_b)

    pltpu.make_async_copy(table_hbm.at[idx_v.at[0]], buf_a, sem_a).start()

    def transpose_block(l, buf):
        rows = []
        cols = []
        msks = []
        for g in range(_BW // 16):
            rowg = lax.iota(jnp.int32, 16) + (16 * g)
            parg = par_v[l, pl.ds(16 * g, 16)]
            mg = msk_v[l, pl.ds(16 * g, 16)]
            rows.append(rowg)
            cols.append(parg * _DIM)
            msks.append(mg)

        def dstep(d, carry):
            for g in range(_BW // 16):
                v = plsc.load_gather(buf, [rows[g], cols[g] + d])
                outb[d, pl.ds(16 * g, 16)] = v * msks[g]
            return carry

        lax.fori_loop(0, _DIM, dstep, 0, unroll=2)

    def step(k, carry):
        for par in (0, 1):
            l = 2 * k + par
            buf = bufs[par]
            sem = sems[par]
            pltpu.make_async_copy(table_hbm.at[idx_v.at[l]], buf, sem).wait()
            nxt = l + 1

            @pl.when(nxt < _L)
            def _():
                pltpu.make_async_copy(
                    table_hbm.at[idx_v.at[nxt]], bufs[1 - par], sems[1 - par]
                ).start()

            transpose_block(l, buf)
            pltpu.sync_copy(
                outb, out_hbm.at[pl.ds(l * _DIM, _DIM), pl.ds(col0, _BW)]
            )
        return carry

    lax.fori_loop(0, _L // 2, step, 0)


_sc_gather = functools.partial(
    pl.kernel,
    out_type=jax.ShapeDtypeStruct((_L * _DIM, _B), jnp.float32),
    mesh=plsc.VectorSubcoreMesh(core_axis_name="c", subcore_axis_name="s"),
    scratch_types=[
        pltpu.VMEM((_L, _BW), jnp.int32),
        pltpu.VMEM((_L, _BW), jnp.int32),
        pltpu.VMEM((_L, _BW), jnp.float32),
        pltpu.VMEM((_BW, 2 * _DIM), jnp.float32),
        pltpu.VMEM((_BW, 2 * _DIM), jnp.float32),
        pltpu.VMEM((_DIM, _BW), jnp.float32),
        pltpu.SemaphoreType.DMA,
        pltpu.SemaphoreType.DMA,
    ],
    compiler_params=pltpu.CompilerParams(
        use_tc_tiling_on_sc=True, needs_layout_passes=False
    ),
)(_sc_body)


def kernel(tokens, mask, embed_table, W, b):
    tok = tokens.astype(jnp.int32)
    idxT = (((tok >> 14) << 13) | (tok & 8191)).T
    parT = ((tok >> 13) & 1).T
    maskT = mask.astype(jnp.float32).T
    tableT = embed_table.T
    p_tab = _tc_prep(tableT, W, b.reshape(1, _DIM))
    outp = _sc_gather(idxT, parT, maskT, p_tab)
    out = outp.reshape(_L, _DIM, _B).transpose(2, 0, 1)
    return out, mask


# speedup vs baseline: 1.4529x; 1.4529x over previous
import functools

import jax
import jax.numpy as jnp
from jax import lax
from jax.experimental import pallas as pl
from jax.experimental.pallas import tpu as pltpu
from jax.experimental.pallas import tpu_sc as plsc

_B = 4096
_L = 50
_DIM = 64
_N = _B * _L
_V = 1000000

_TBLK = 16384
_NBLK = (_V + _TBLK - 1) // _TBLK
_VP = _NBLK * (_TBLK // 2)

_NW = 32
_BW = _B // _NW


def _tc_prep_body(x_ref, w_ref, b_ref, o_ref):
    x = x_ref[...]
    cdims = (((0,), (1,)), ((), ()))
    a = lax.dot_general(x[:, :_TBLK // 2], w_ref[...], cdims,
                        preferred_element_type=jnp.float32)
    c = lax.dot_general(x[:, _TBLK // 2:], w_ref[...], cdims,
                        preferred_element_type=jnp.float32)
    bb = b_ref[...]
    o_ref[...] = jnp.concatenate([a + bb, c + bb], axis=1)


_tc_prep = pl.pallas_call(
    _tc_prep_body,
    grid=(_NBLK,),
    in_specs=[
        pl.BlockSpec((_DIM, _TBLK), lambda i: (0, i)),
        pl.BlockSpec((_DIM, _DIM), lambda i: (0, 0)),
        pl.BlockSpec((1, _DIM), lambda i: (0, 0)),
    ],
    out_specs=pl.BlockSpec((_TBLK // 2, 2 * _DIM), lambda i: (i, 0)),
    out_shape=jax.ShapeDtypeStruct((_VP, 2 * _DIM), jnp.float32),
)


def _sc_gather_body(idx_hbm, table_hbm, out_hbm, idx_v, rows_a, rows_b, sem_a, sem_b):
    wid = lax.axis_index("s") * 2 + lax.axis_index("c")
    col0 = wid * _BW
    pltpu.sync_copy(idx_hbm.at[:, pl.ds(col0, _BW)], idx_v)

    bufs = (rows_a, rows_b)
    sems = (sem_a, sem_b)

    pltpu.make_async_copy(table_hbm.at[idx_v.at[0]], rows_a, sem_a).start()

    def step(k, carry):
        for par in (0, 1):
            l = 2 * k + par
            buf = bufs[par]
            sem = sems[par]
            pltpu.make_async_copy(table_hbm.at[idx_v.at[l]], buf, sem).wait()
            nxt = l + 1

            @pl.when(nxt < _L)
            def _():
                pltpu.make_async_copy(
                    table_hbm.at[idx_v.at[nxt]], bufs[1 - par], sems[1 - par]
                ).start()

            pltpu.sync_copy(buf, out_hbm.at[pl.ds(l * _B + col0, _BW)])
        return carry

    lax.fori_loop(0, _L // 2, step, 0)


_sc_gather = functools.partial(
    pl.kernel,
    out_type=jax.ShapeDtypeStruct((_N, 2 * _DIM), jnp.float32),
    mesh=plsc.VectorSubcoreMesh(core_axis_name="c", subcore_axis_name="s"),
    scratch_types=[
        pltpu.VMEM((_L, _BW), jnp.int32),
        pltpu.VMEM((_BW, 2 * _DIM), jnp.float32),
        pltpu.VMEM((_BW, 2 * _DIM), jnp.float32),
        pltpu.SemaphoreType.DMA,
        pltpu.SemaphoreType.DMA,
    ],
    compiler_params=pltpu.CompilerParams(use_tc_tiling_on_sc=True),
)(_sc_gather_body)


def _tc_sel_body(x_ref, par_ref, m_ref, il_ref, ir_ref, o_ref):
    x = x_ref[...]
    cdims = (((1,), (1,)), ((), ()))
    a = lax.dot_general(il_ref[...], x, cdims,
                        preferred_element_type=jnp.float32)
    c = lax.dot_general(ir_ref[...], x, cdims,
                        preferred_element_type=jnp.float32)
    par = par_ref[0] != 0
    sel = jnp.where(par, c, a)
    o_ref[...] = sel * m_ref[0].astype(jnp.float32)


_tc_sel = pl.pallas_call(
    _tc_sel_body,
    grid=(_L,),
    in_specs=[
        pl.BlockSpec((_B, 2 * _DIM), lambda l: (l, 0)),
        pl.BlockSpec((1, 1, _B), lambda l: (l, 0, 0)),
        pl.BlockSpec((1, 1, _B), lambda l: (l, 0, 0)),
        pl.BlockSpec((_DIM, 2 * _DIM), lambda l: (0, 0)),
        pl.BlockSpec((_DIM, 2 * _DIM), lambda l: (0, 0)),
    ],
    out_specs=pl.BlockSpec((_DIM, _B), lambda l: (l, 0)),
    out_shape=jax.ShapeDtypeStruct((_L * _DIM, _B), jnp.float32),
)


def kernel(tokens, mask, embed_table, W, b):
    tok = tokens.astype(jnp.int32)
    idxT = (((tok >> 14) << 13) | (tok & 8191)).T
    parT = ((tok >> 13) & 1).T.reshape(_L, 1, _B)
    maskT = mask.T.reshape(_L, 1, _B)
    tableT = embed_table.T
    p_tab = _tc_prep(tableT, W, b.reshape(1, _DIM))
    gathered = _sc_gather(idxT, p_tab)
    eye = jnp.eye(_DIM, dtype=jnp.float32)
    zero = jnp.zeros((_DIM, _DIM), jnp.float32)
    il = jnp.concatenate([eye, zero], axis=1)
    ir = jnp.concatenate([zero, eye], axis=1)
    outp = _tc_sel(gathered, parT, maskT, il, ir)
    out = outp.reshape(_L, _DIM, _B).transpose(2, 0, 1)
    return out, mask


# speedup vs baseline: 1.6018x; 1.1024x over previous
import functools

import jax
import jax.numpy as jnp
from jax import lax
from jax.experimental import pallas as pl
from jax.experimental.pallas import tpu as pltpu
from jax.experimental.pallas import tpu_sc as plsc

_B = 4096
_L = 50
_DIM = 64
_N = _B * _L
_V = 1000000

_TBLK = 32768
_NBLK = (_V + _TBLK - 1) // _TBLK
_VP = _NBLK * (_TBLK // 2)

_NW = 32
_BW = _B // _NW


def _tc_prep_body(x_ref, w_ref, b_ref, o_ref):
    x = x_ref[...]
    cdims = (((0,), (1,)), ((), ()))
    a = lax.dot_general(x[:, :_TBLK // 2], w_ref[...], cdims,
                        preferred_element_type=jnp.float32)
    c = lax.dot_general(x[:, _TBLK // 2:], w_ref[...], cdims,
                        preferred_element_type=jnp.float32)
    bb = b_ref[...]
    o_ref[...] = jnp.concatenate([a + bb, c + bb], axis=1)


_tc_prep = pl.pallas_call(
    _tc_prep_body,
    grid=(_NBLK,),
    in_specs=[
        pl.BlockSpec((_DIM, _TBLK), lambda i: (0, i)),
        pl.BlockSpec((_DIM, _DIM), lambda i: (0, 0)),
        pl.BlockSpec((1, _DIM), lambda i: (0, 0)),
    ],
    out_specs=pl.BlockSpec((_TBLK // 2, 2 * _DIM), lambda i: (i, 0)),
    out_shape=jax.ShapeDtypeStruct((_VP, 2 * _DIM), jnp.float32),
)


def _sc_gather_body(idx_hbm, table_hbm, out_hbm, idx_v, rows_a, rows_b, sem_a, sem_b):
    wid = lax.axis_index("s") * 2 + lax.axis_index("c")
    col0 = wid * _BW
    pltpu.sync_copy(idx_hbm.at[:, pl.ds(col0, _BW)], idx_v)

    bufs = (rows_a, rows_b)
    sems = (sem_a, sem_b)

    pltpu.make_async_copy(table_hbm.at[idx_v.at[0]], rows_a, sem_a).start()

    def step(k, carry):
        for par in (0, 1):
            l = 2 * k + par
            buf = bufs[par]
            sem = sems[par]
            pltpu.make_async_copy(table_hbm.at[idx_v.at[l]], buf, sem).wait()
            nxt = l + 1

            @pl.when(nxt < _L)
            def _():
                pltpu.make_async_copy(
                    table_hbm.at[idx_v.at[nxt]], bufs[1 - par], sems[1 - par]
                ).start()

            pltpu.sync_copy(buf, out_hbm.at[pl.ds(l * _B + col0, _BW)])
        return carry

    lax.fori_loop(0, _L // 2, step, 0)


_sc_gather = functools.partial(
    pl.kernel,
    out_type=jax.ShapeDtypeStruct((_N, 2 * _DIM), jnp.float32),
    mesh=plsc.VectorSubcoreMesh(core_axis_name="c", subcore_axis_name="s"),
    scratch_types=[
        pltpu.VMEM((_L, _BW), jnp.int32),
        pltpu.VMEM((_BW, 2 * _DIM), jnp.float32),
        pltpu.VMEM((_BW, 2 * _DIM), jnp.float32),
        pltpu.SemaphoreType.DMA,
        pltpu.SemaphoreType.DMA,
    ],
    compiler_params=pltpu.CompilerParams(use_tc_tiling_on_sc=True),
)(_sc_gather_body)


_LSEL = 2


def _tc_sel_body(x_ref, par_ref, m_ref, il_ref, ir_ref, o_ref):
    x = x_ref[...]
    cdims = (((1,), (1,)), ((), ()))
    outs = []
    for h in range(_LSEL):
        xs = x[h * _B:(h + 1) * _B]
        a = lax.dot_general(il_ref[...], xs, cdims,
                            preferred_element_type=jnp.float32)
        c = lax.dot_general(ir_ref[...], xs, cdims,
                            preferred_element_type=jnp.float32)
        par = par_ref[h] != 0
        sel = jnp.where(par, c, a)
        outs.append(sel * m_ref[h].astype(jnp.float32))
    o_ref[...] = jnp.concatenate(outs, axis=0)


_tc_sel = pl.pallas_call(
    _tc_sel_body,
    grid=(_L // _LSEL,),
    in_specs=[
        pl.BlockSpec((_LSEL * _B, 2 * _DIM), lambda l: (l, 0)),
        pl.BlockSpec((_LSEL, 1, _B), lambda l: (l, 0, 0)),
        pl.BlockSpec((_LSEL, 1, _B), lambda l: (l, 0, 0)),
        pl.BlockSpec((_DIM, 2 * _DIM), lambda l: (0, 0)),
        pl.BlockSpec((_DIM, 2 * _DIM), lambda l: (0, 0)),
    ],
    out_specs=pl.BlockSpec((_LSEL * _DIM, _B), lambda l: (l, 0)),
    out_shape=jax.ShapeDtypeStruct((_L * _DIM, _B), jnp.float32),
)


def kernel(tokens, mask, embed_table, W, b):
    tok = tokens.astype(jnp.int32)
    idxT = (((tok >> 15) << 14) | (tok & 16383)).T
    parT = ((tok >> 14) & 1).T.reshape(_L, 1, _B)
    maskT = mask.T.reshape(_L, 1, _B)
    tableT = embed_table.T
    p_tab = _tc_prep(tableT, W, b.reshape(1, _DIM))
    gathered = _sc_gather(idxT, p_tab)
    eye = jnp.eye(_DIM, dtype=jnp.float32)
    zero = jnp.zeros((_DIM, _DIM), jnp.float32)
    il = jnp.concatenate([eye, zero], axis=1)
    ir = jnp.concatenate([zero, eye], axis=1)
    outp = _tc_sel(gathered, parT, maskT, il, ir)
    out = outp.reshape(_L, _DIM, _B).transpose(2, 0, 1)
    return out, mask


# speedup vs baseline: 1.8247x; 1.1392x over previous
import functools

import jax
import jax.numpy as jnp
from jax import lax
from jax.experimental import pallas as pl
from jax.experimental.pallas import tpu as pltpu
from jax.experimental.pallas import tpu_sc as plsc

_B = 4096
_L = 50
_DIM = 64
_N = _B * _L
_V = 1000000

_TBLK = 32768
_NBLK = (_V + _TBLK - 1) // _TBLK
_VP = _NBLK * (_TBLK // 2)

_NW = 32
_BW = _B // _NW


def _tc_prep_body(x_ref, w_ref, b_ref, o_ref):
    x = x_ref[...].astype(jnp.bfloat16)
    w = w_ref[...].astype(jnp.bfloat16)
    cdims = (((0,), (1,)), ((), ()))
    a = lax.dot_general(x[:, :_TBLK // 2], w, cdims,
                        preferred_element_type=jnp.float32)
    c = lax.dot_general(x[:, _TBLK // 2:], w, cdims,
                        preferred_element_type=jnp.float32)
    bb = b_ref[...]
    o_ref[...] = jnp.concatenate([a + bb, c + bb], axis=1)


_tc_prep = pl.pallas_call(
    _tc_prep_body,
    grid=(_NBLK,),
    in_specs=[
        pl.BlockSpec((_DIM, _TBLK), lambda i: (0, i)),
        pl.BlockSpec((_DIM, _DIM), lambda i: (0, 0)),
        pl.BlockSpec((1, _DIM), lambda i: (0, 0)),
    ],
    out_specs=pl.BlockSpec((_TBLK // 2, 2 * _DIM), lambda i: (i, 0)),
    out_shape=jax.ShapeDtypeStruct((_VP, 2 * _DIM), jnp.float32),
)


_NBUF = 4


def _sc_gather_body(idx_hbm, table_hbm, out_hbm, idx_v, *rest):
    bufs = rest[:_NBUF]
    gsems = rest[_NBUF:2 * _NBUF]
    wsems = rest[2 * _NBUF:3 * _NBUF]
    wid = lax.axis_index("s") * 2 + lax.axis_index("c")
    col0 = wid * _BW
    pltpu.sync_copy(idx_hbm.at[:, pl.ds(col0, _BW)], idx_v)

    def gather_cp(l, j):
        return pltpu.make_async_copy(table_hbm.at[idx_v.at[l]], bufs[j], gsems[j])

    def write_cp(l, j):
        return pltpu.make_async_copy(
            bufs[j], out_hbm.at[pl.ds(l * _B + col0, _BW)], wsems[j]
        )

    for j in range(_NBUF - 1):
        gather_cp(j, j).start()

    def slot(l, par, guard_prev, guard_next):
        gather_cp(l, par).wait()
        write_cp(l, par).start()
        prev = (par + _NBUF - 1) % _NBUF
        if guard_prev:

            @pl.when(l > 0)
            def _():
                write_cp(l - 1, prev).wait()

        else:
            write_cp(l - 1, prev).wait()
        nxt = l + _NBUF - 1
        if guard_next:

            @pl.when(nxt < _L)
            def _():
                gather_cp(nxt, prev).start()

        else:
            gather_cp(nxt, prev).start()

    def step(k, carry):
        for par in range(_NBUF):
            slot(_NBUF * k + par, par, guard_prev=True, guard_next=True)
        return carry

    nfull = (_L // _NBUF) * _NBUF
    lax.fori_loop(0, _L // _NBUF, step, 0)
    for l in range(nfull, _L):
        par = l % _NBUF
        gather_cp(l, par).wait()
        write_cp(l, par).start()
        write_cp(l - 1, (par + _NBUF - 1) % _NBUF).wait()
    write_cp(_L - 1, (_L - 1) % _NBUF).wait()


_sc_gather = functools.partial(
    pl.kernel,
    out_type=jax.ShapeDtypeStruct((_N, 2 * _DIM), jnp.float32),
    mesh=plsc.VectorSubcoreMesh(core_axis_name="c", subcore_axis_name="s"),
    scratch_types=[
        pltpu.VMEM((_L, _BW), jnp.int32),
    ]
    + [pltpu.VMEM((_BW, 2 * _DIM), jnp.float32)] * _NBUF
    + [pltpu.SemaphoreType.DMA] * (2 * _NBUF),
    compiler_params=pltpu.CompilerParams(use_tc_tiling_on_sc=True),
)(_sc_gather_body)


_LSEL = 2


def _tc_sel_body(x_ref, par_ref, m_ref, il_ref, ir_ref, o_ref):
    x = x_ref[...]
    cdims = (((1,), (1,)), ((), ()))
    outs = []
    for h in range(_LSEL):
        xs = x[h * _B:(h + 1) * _B]
        a = lax.dot_general(il_ref[...], xs, cdims,
                            preferred_element_type=jnp.float32)
        c = lax.dot_general(ir_ref[...], xs, cdims,
                            preferred_element_type=jnp.float32)
        par = par_ref[h] != 0
        sel = jnp.where(par, c, a)
        outs.append(sel * m_ref[h].astype(jnp.float32))
    o_ref[...] = jnp.concatenate(outs, axis=0)


_tc_sel = pl.pallas_call(
    _tc_sel_body,
    grid=(_L // _LSEL,),
    in_specs=[
        pl.BlockSpec((_LSEL * _B, 2 * _DIM), lambda l: (l, 0)),
        pl.BlockSpec((_LSEL, 1, _B), lambda l: (l, 0, 0)),
        pl.BlockSpec((_LSEL, 1, _B), lambda l: (l, 0, 0)),
        pl.BlockSpec((_DIM, 2 * _DIM), lambda l: (0, 0)),
        pl.BlockSpec((_DIM, 2 * _DIM), lambda l: (0, 0)),
    ],
    out_specs=pl.BlockSpec((_LSEL * _DIM, _B), lambda l: (l, 0)),
    out_shape=jax.ShapeDtypeStruct((_L * _DIM, _B), jnp.float32),
)


def kernel(tokens, mask, embed_table, W, b):
    tok = tokens.astype(jnp.int32)
    idxT = (((tok >> 15) << 14) | (tok & 16383)).T
    parT = ((tok >> 14) & 1).T.reshape(_L, 1, _B)
    maskT = mask.T.reshape(_L, 1, _B)
    tableT = embed_table.T
    p_tab = _tc_prep(tableT, W, b.reshape(1, _DIM))
    gathered = _sc_gather(idxT, p_tab)
    eye = jnp.eye(_DIM, dtype=jnp.float32)
    zero = jnp.zeros((_DIM, _DIM), jnp.float32)
    il = jnp.concatenate([eye, zero], axis=1)
    ir = jnp.concatenate([zero, eye], axis=1)
    outp = _tc_sel(gathered, parT, maskT, il, ir)
    out = outp.reshape(_L, _DIM, _B).transpose(2, 0, 1)
    return out, mask


# speedup vs baseline: 1.8927x; 1.0373x over previous
import functools

import jax
import jax.numpy as jnp
from jax import lax
from jax.experimental import pallas as pl
from jax.experimental.pallas import tpu as pltpu
from jax.experimental.pallas import tpu_sc as plsc

_B = 4096
_L = 50
_DIM = 64
_N = _B * _L
_V = 1000000

_TBLK = 32768
_NBLK = (_V + _TBLK - 1) // _TBLK
_QUART = _TBLK // 4
_VP = _NBLK * _QUART

_NW = 32
_BW = _B // _NW


def _tc_prep_body(x_ref, w_ref, b_ref, o_ref):
    x = x_ref[...].astype(jnp.bfloat16)
    w = w_ref[...].astype(jnp.bfloat16)
    bb = b_ref[...]
    cdims = (((0,), (1,)), ((), ()))
    quads = []
    for q in range(4):
        p = lax.dot_general(x[:, q * _QUART:(q + 1) * _QUART], w, cdims,
                            preferred_element_type=jnp.float32)
        pb = (p + bb).astype(jnp.bfloat16)
        quads.append(lax.bitcast_convert_type(pb, jnp.uint16).astype(jnp.uint32))
    lo = quads[0] | (quads[1] << 16)
    hi = quads[2] | (quads[3] << 16)
    o_ref[...] = lax.bitcast_convert_type(
        jnp.concatenate([lo, hi], axis=1), jnp.int32
    )


_tc_prep = pl.pallas_call(
    _tc_prep_body,
    grid=(_NBLK,),
    in_specs=[
        pl.BlockSpec((_DIM, _TBLK), lambda i: (0, i)),
        pl.BlockSpec((_DIM, _DIM), lambda i: (0, 0)),
        pl.BlockSpec((1, _DIM), lambda i: (0, 0)),
    ],
    out_specs=pl.BlockSpec((_QUART, 2 * _DIM), lambda i: (i, 0)),
    out_shape=jax.ShapeDtypeStruct((_VP, 2 * _DIM), jnp.int32),
)


_NBUF = 4


def _sc_gather_body(idx_hbm, table_hbm, out_hbm, idx_v, *rest):
    bufs = rest[:_NBUF]
    gsems = rest[_NBUF:2 * _NBUF]
    wsems = rest[2 * _NBUF:3 * _NBUF]
    wid = lax.axis_index("s") * 2 + lax.axis_index("c")
    col0 = wid * _BW
    pltpu.sync_copy(idx_hbm.at[:, pl.ds(col0, _BW)], idx_v)

    def gather_cp(l, j):
        return pltpu.make_async_copy(table_hbm.at[idx_v.at[l]], bufs[j], gsems[j])

    def write_cp(l, j):
        return pltpu.make_async_copy(
            bufs[j], out_hbm.at[pl.ds(l * _B + col0, _BW)], wsems[j]
        )

    for j in range(_NBUF - 1):
        gather_cp(j, j).start()

    def slot(l, par, guard_prev, guard_next):
        gather_cp(l, par).wait()
        write_cp(l, par).start()
        prev = (par + _NBUF - 1) % _NBUF
        if guard_prev:

            @pl.when(l > 0)
            def _():
                write_cp(l - 1, prev).wait()

        else:
            write_cp(l - 1, prev).wait()
        nxt = l + _NBUF - 1
        if guard_next:

            @pl.when(nxt < _L)
            def _():
                gather_cp(nxt, prev).start()

        else:
            gather_cp(nxt, prev).start()

    def step(k, carry):
        for par in range(_NBUF):
            slot(_NBUF * k + par, par, guard_prev=True, guard_next=True)
        return carry

    nfull = (_L // _NBUF) * _NBUF
    lax.fori_loop(0, _L // _NBUF, step, 0)
    for l in range(nfull, _L):
        par = l % _NBUF
        gather_cp(l, par).wait()
        write_cp(l, par).start()
        write_cp(l - 1, (par + _NBUF - 1) % _NBUF).wait()
    write_cp(_L - 1, (_L - 1) % _NBUF).wait()


_sc_gather = functools.partial(
    pl.kernel,
    out_type=jax.ShapeDtypeStruct((_N, 2 * _DIM), jnp.int32),
    mesh=plsc.VectorSubcoreMesh(core_axis_name="c", subcore_axis_name="s"),
    scratch_types=[
        pltpu.VMEM((_L, _BW), jnp.int32),
    ]
    + [pltpu.VMEM((_BW, 2 * _DIM), jnp.int32)] * _NBUF
    + [pltpu.SemaphoreType.DMA] * (2 * _NBUF),
    compiler_params=pltpu.CompilerParams(use_tc_tiling_on_sc=True),
)(_sc_gather_body)


_LSEL = 2


def _tc_sel_body(x_ref, par_ref, m_ref, il_ref, ir_ref, o_ref):
    x = x_ref[...]
    cdims = (((1,), (1,)), ((), ()))
    lo = lax.bitcast_convert_type(x << 16, jnp.float32)
    hi = lax.bitcast_convert_type(
        x & jnp.int32(-65536), jnp.float32
    )
    outs = []
    for h in range(_LSEL):
        sl = slice(h * _B, (h + 1) * _B)
        cands = []
        for plane in (lo, hi):
            xs = plane[sl]
            a = lax.dot_general(il_ref[...], xs, cdims,
                                preferred_element_type=jnp.float32)
            c = lax.dot_general(ir_ref[...], xs, cdims,
                                preferred_element_type=jnp.float32)
            cands.append((a, c))
        par = par_ref[h]
        s_lo = jnp.where((par & 1) != 0, cands[1][0], cands[0][0])
        s_hi = jnp.where((par & 1) != 0, cands[1][1], cands[0][1])
        sel = jnp.where((par & 2) != 0, s_hi, s_lo)
        outs.append(sel * m_ref[h].astype(jnp.float32))
    o_ref[...] = jnp.concatenate(outs, axis=0)


_tc_sel = pl.pallas_call(
    _tc_sel_body,
    grid=(_L // _LSEL,),
    in_specs=[
        pl.BlockSpec((_LSEL * _B, 2 * _DIM), lambda l: (l, 0)),
        pl.BlockSpec((_LSEL, 1, _B), lambda l: (l, 0, 0)),
        pl.BlockSpec((_LSEL, 1, _B), lambda l: (l, 0, 0)),
        pl.BlockSpec((_DIM, 2 * _DIM), lambda l: (0, 0)),
        pl.BlockSpec((_DIM, 2 * _DIM), lambda l: (0, 0)),
    ],
    out_specs=pl.BlockSpec((_LSEL * _DIM, _B), lambda l: (l, 0)),
    out_shape=jax.ShapeDtypeStruct((_L * _DIM, _B), jnp.float32),
)


def kernel(tokens, mask, embed_table, W, b):
    tok = tokens.astype(jnp.int32)
    idxT = (((tok >> 15) << 13) | (tok & 8191)).T
    parT = ((tok >> 13) & 3).T.reshape(_L, 1, _B)
    maskT = mask.T.reshape(_L, 1, _B)
    tableT = embed_table.T
    p_tab = _tc_prep(tableT, W, b.reshape(1, _DIM))
    gathered = _sc_gather(idxT, p_tab)
    eye = jnp.eye(_DIM, dtype=jnp.float32)
    zero = jnp.zeros((_DIM, _DIM), jnp.float32)
    il = jnp.concatenate([eye, zero], axis=1)
    ir = jnp.concatenate([zero, eye], axis=1)
    outp = _tc_sel(gathered, parT, maskT, il, ir)
    out = outp.reshape(_L, _DIM, _B).transpose(2, 0, 1)
    return out, mask


# speedup vs baseline: 1.9576x; 1.0343x over previous
import functools

import jax
import jax.numpy as jnp
from jax import lax
from jax.experimental import pallas as pl
from jax.experimental.pallas import tpu as pltpu
from jax.experimental.pallas import tpu_sc as plsc

_B = 4096
_L = 50
_DIM = 64
_N = _B * _L
_V = 1000000

_TBLK = 32768
_NBLK = (_V + _TBLK - 1) // _TBLK
_QUART = _TBLK // 4
_VP = _NBLK * _QUART

_NW = 32
_BW = _B // _NW


def _tc_prep_body(x_ref, w_ref, b_ref, o_ref):
    x = x_ref[...].astype(jnp.bfloat16)
    w = w_ref[...].astype(jnp.bfloat16)
    bb = b_ref[...]
    cdims = (((0,), (1,)), ((), ()))
    quads = []
    for q in range(4):
        p = lax.dot_general(x[:, q * _QUART:(q + 1) * _QUART], w, cdims,
                            preferred_element_type=jnp.float32)
        pb = (p + bb).astype(jnp.bfloat16)
        quads.append(lax.bitcast_convert_type(pb, jnp.uint16).astype(jnp.uint32))
    lo = quads[0] | (quads[1] << 16)
    hi = quads[2] | (quads[3] << 16)
    o_ref[...] = lax.bitcast_convert_type(
        jnp.concatenate([lo, hi], axis=1), jnp.int32
    )


_tc_prep = pl.pallas_call(
    _tc_prep_body,
    grid=(_NBLK,),
    in_specs=[
        pl.BlockSpec((_DIM, _TBLK), lambda i: (0, i)),
        pl.BlockSpec((_DIM, _DIM), lambda i: (0, 0)),
        pl.BlockSpec((1, _DIM), lambda i: (0, 0)),
    ],
    out_specs=pl.BlockSpec((_QUART, 2 * _DIM), lambda i: (i, 0)),
    out_shape=jax.ShapeDtypeStruct((_VP, 2 * _DIM), jnp.int32),
)


_NBUF = 4


def _make_sc_gather(l0, nl):
    def body(idx_hbm, table_hbm, out_hbm, idx_v, *rest):
        bufs = rest[:_NBUF]
        gsems = rest[_NBUF:2 * _NBUF]
        wsems = rest[2 * _NBUF:3 * _NBUF]
        wid = lax.axis_index("s") * 2 + lax.axis_index("c")
        col0 = wid * _BW
        pltpu.sync_copy(idx_hbm.at[:, pl.ds(col0, _BW)], idx_v)

        def gather_cp(l, j):
            return pltpu.make_async_copy(
                table_hbm.at[idx_v.at[l0 + l]], bufs[j], gsems[j]
            )

        def write_cp(l, j):
            return pltpu.make_async_copy(
                bufs[j], out_hbm.at[pl.ds(l * _B + col0, _BW)], wsems[j]
            )

        for j in range(_NBUF - 1):
            gather_cp(j, j).start()

        def step(k, carry):
            for par in range(_NBUF):
                l = _NBUF * k + par
                gather_cp(l, par).wait()
                write_cp(l, par).start()
                prev = (par + _NBUF - 1) % _NBUF

                @pl.when(l > 0)
                def _():
                    write_cp(l - 1, prev).wait()

                nxt = l + _NBUF - 1

                @pl.when(nxt < nl)
                def _():
                    gather_cp(nxt, prev).start()

            return carry

        nfull = (nl // _NBUF) * _NBUF
        lax.fori_loop(0, nl // _NBUF, step, 0)
        for l in range(nfull, nl):
            par = l % _NBUF
            gather_cp(l, par).wait()
            write_cp(l, par).start()
            write_cp(l - 1, (par + _NBUF - 1) % _NBUF).wait()
        write_cp(nl - 1, (nl - 1) % _NBUF).wait()

    return functools.partial(
        pl.kernel,
        out_type=jax.ShapeDtypeStruct((nl * _B, 2 * _DIM), jnp.int32),
        mesh=plsc.VectorSubcoreMesh(core_axis_name="c", subcore_axis_name="s"),
        scratch_types=[
            pltpu.VMEM((_L, _BW), jnp.int32),
        ]
        + [pltpu.VMEM((_BW, 2 * _DIM), jnp.int32)] * _NBUF
        + [pltpu.SemaphoreType.DMA] * (2 * _NBUF),
        compiler_params=pltpu.CompilerParams(use_tc_tiling_on_sc=True),
    )(body)


_LSPLIT = 26
_sc_gather_a = _make_sc_gather(0, _LSPLIT)
_sc_gather_b = _make_sc_gather(_LSPLIT, _L - _LSPLIT)


_LSEL = 2


def _sel_math(x, par_ref, m_ref, il_ref, ir_ref):
    cdims = (((1,), (1,)), ((), ()))
    lo = lax.bitcast_convert_type(x << 16, jnp.float32)
    hi = lax.bitcast_convert_type(
        x & jnp.int32(-65536), jnp.float32
    )
    outs = []
    for h in range(_LSEL):
        sl = slice(h * _B, (h + 1) * _B)
        cands = []
        for plane in (lo, hi):
            xs = plane[sl]
            a = lax.dot_general(il_ref[...], xs, cdims,
                                preferred_element_type=jnp.float32)
            c = lax.dot_general(ir_ref[...], xs, cdims,
                                preferred_element_type=jnp.float32)
            cands.append((a, c))
        par = par_ref[h]
        s_lo = jnp.where((par & 1) != 0, cands[1][0], cands[0][0])
        s_hi = jnp.where((par & 1) != 0, cands[1][1], cands[0][1])
        sel = jnp.where((par & 2) != 0, s_hi, s_lo)
        outs.append(sel * m_ref[h].astype(jnp.float32))
    return jnp.concatenate(outs, axis=0)


def _tc_sel_a_body(x_ref, par_ref, m_ref, il_ref, ir_ref, o_ref):
    o_ref[...] = _sel_math(x_ref[...], par_ref, m_ref, il_ref, ir_ref)


def _tc_sel_b_body(x_ref, par_ref, m_ref, il_ref, ir_ref, prev_ref, o_ref):
    del prev_ref
    o_ref[...] = _sel_math(x_ref[...], par_ref, m_ref, il_ref, ir_ref)


def _make_tc_sel(l0, nl, aliased):
    g0 = l0 // _LSEL
    in_specs = [
        pl.BlockSpec((_LSEL * _B, 2 * _DIM), lambda l: (l, 0)),
        pl.BlockSpec((_LSEL, 1, _B), lambda l: (l + g0, 0, 0)),
        pl.BlockSpec((_LSEL, 1, _B), lambda l: (l + g0, 0, 0)),
        pl.BlockSpec((_DIM, 2 * _DIM), lambda l: (0, 0)),
        pl.BlockSpec((_DIM, 2 * _DIM), lambda l: (0, 0)),
    ]
    kwargs = {}
    if aliased:
        in_specs.append(pl.BlockSpec(memory_space=pl.ANY))
        kwargs["input_output_aliases"] = {5: 0}
    return pl.pallas_call(
        _tc_sel_b_body if aliased else _tc_sel_a_body,
        grid=(nl // _LSEL,),
        in_specs=in_specs,
        out_specs=pl.BlockSpec((_LSEL * _DIM, _B), lambda l: (l + g0, 0)),
        out_shape=jax.ShapeDtypeStruct((_L * _DIM, _B), jnp.float32),
        **kwargs,
    )


_tc_sel_a = _make_tc_sel(0, _LSPLIT, aliased=False)
_tc_sel_b = _make_tc_sel(_LSPLIT, _L - _LSPLIT, aliased=True)


def kernel(tokens, mask, embed_table, W, b):
    tok = tokens.astype(jnp.int32)
    idxT = (((tok >> 15) << 13) | (tok & 8191)).T
    parT = ((tok >> 13) & 3).T.reshape(_L, 1, _B)
    maskT = mask.T.reshape(_L, 1, _B)
    tableT = embed_table.T
    p_tab = _tc_prep(tableT, W, b.reshape(1, _DIM))
    g_a = _sc_gather_a(idxT, p_tab)
    g_b = _sc_gather_b(idxT, p_tab)
    eye = jnp.eye(_DIM, dtype=jnp.float32)
    zero = jnp.zeros((_DIM, _DIM), jnp.float32)
    il = jnp.concatenate([eye, zero], axis=1)
    ir = jnp.concatenate([zero, eye], axis=1)
    outp_a = _tc_sel_a(g_a, parT, maskT, il, ir)
    outp = _tc_sel_b(g_b, parT, maskT, il, ir, outp_a)
    out = outp.reshape(_L, _DIM, _B).transpose(2, 0, 1)
    return out, mask
